# Initial kernel scaffold; baseline (speedup 1.0000x reference)
import jax
import jax.numpy as jnp
from jax.experimental import pallas as pl


def kernel(x, params):
    raise NotImplementedError("write your pallas kernel here")


# speedup vs baseline: 1.2047x; 1.2047x over previous
import jax
import jax.numpy as jnp
from jax.experimental import pallas as pl
from jax.experimental.pallas import tpu as pltpu

B, S, H, K = 512, 64, 256, 1024
H2, H4 = H // 2, H // 4


def _mm(a, w):
    return jax.lax.dot_general(a, w, (((1,), (0,)), ((), ())),
                               preferred_element_type=jnp.float32)


def _ln(h, g, b):
    m = jnp.mean(h, axis=-1, keepdims=True)
    v = jnp.mean((h - m) ** 2, axis=-1, keepdims=True)
    return (h - m) / jnp.sqrt(v + 1e-5) * g + b


def _vq_step(x_ref, cb_ref,
             eW1, eb1, eg1, ebn1, eW2, eb2, eg2, ebn2, eW3, eb3,
             er0W1, er0b1, er0W2, er0b2, er1W1, er1b1, er1W2, er1b2,
             dr0W1, dr0b1, dr0W2, dr0b2, dr1W1, dr1b1, dr1W2, dr1b2,
             dW1, db1, dg1, dbn1, dW2, db2, dg2, dbn2, dW3, db3,
             out_ref, oh_ref, loss_ref):
    s = pl.program_id(0)
    xrow = x_ref[0]
    xcol = xrow.T

    h = xcol * eW1[...] + eb1[...]
    h = jnp.tanh(_ln(h, eg1[...], ebn1[...]))
    h = jnp.tanh(_ln(_mm(h, eW2[...]) + eb2[...], eg2[...], ebn2[...]))
    h = _mm(h, eW3[...]) + eb3[...]
    for W1, b1, W2, b2 in ((er0W1, er0b1, er0W2, er0b2),
                           (er1W1, er1b1, er1W2, er1b2)):
        t = _mm(jax.nn.relu(h), W1[...]) + b1[...]
        t = _mm(jax.nn.relu(t), W2[...]) + b2[...]
        h = h + t
    h = jax.nn.relu(h)

    cb = cb_ref[0]
    x2 = jnp.sum(h * h, axis=-1, keepdims=True)
    e2 = jnp.sum(cb * cb, axis=-1)[None, :]
    xe = jax.lax.dot_general(h, cb, (((1,), (1,)), ((), ())),
                             preferred_element_type=jnp.float32)
    dist = x2 + e2 - 2.0 * xe
    mind = jnp.min(dist, axis=-1, keepdims=True)
    iota = jax.lax.broadcasted_iota(jnp.int32, (B, K), 1)
    idx = jnp.min(jnp.where(dist == mind, iota, K), axis=-1, keepdims=True)
    onehot = (iota == idx).astype(jnp.float32)
    oh_ref[...] = onehot

    q = _mm(onehot, cb)
    commit_part = jnp.sum((q - h) ** 2)
    d = h + (q - h)

    for W1, b1, W2, b2 in ((dr0W1, dr0b1, dr0W2, dr0b2),
                           (dr1W1, dr1b1, dr1W2, dr1b2)):
        t = _mm(jax.nn.relu(d), W1[...]) + b1[...]
        t = _mm(jax.nn.relu(t), W2[...]) + b2[...]
        d = d + t
    d = jax.nn.relu(d)
    d = jnp.tanh(_ln(_mm(d, dW1[...]) + db1[...], dg1[...], dbn1[...]))
    d = jnp.tanh(_ln(_mm(d, dW2[...]) + db2[...], dg2[...], dbn2[...]))
    orow = jax.lax.dot_general(dW3[...], d, (((0,), (1,)), ((), ())),
                               preferred_element_type=jnp.float32) + db3[...]
    out_ref[0] = orow

    recons_part = jnp.sum((orow - xrow) ** 2)
    part = commit_part / (B * S * H) + recons_part / (B * S)

    @pl.when(s == 0)
    def _init():
        loss_ref[...] = jnp.zeros((1, 1), jnp.float32)
    loss_ref[...] += jnp.reshape(part, (1, 1))


def _full(shape):
    nd = len(shape)
    return pl.BlockSpec(shape, lambda s, _nd=nd: (0,) * _nd)


def kernel(x, params):
    p = params
    x_t = x.T.reshape(S, 1, B)

    def w2(a):
        return a.reshape(1, -1) if a.ndim == 1 else a

    flat = [p['eW1'], p['eb1'], p['eg1'], p['ebn1'],
            p['eW2'], p['eb2'], p['eg2'], p['ebn2'],
            p['eW3'], p['eb3']]
    for lay in p['enc_res']:
        flat.extend(lay)
    for lay in p['dec_res']:
        flat.extend(lay)
    flat.extend([p['dW1'], p['db1'], p['dg1'], p['dbn1'],
                 p['dW2'], p['db2'], p['dg2'], p['dbn2'],
                 p['dW3'], p['db3']])
    flat = [w2(a) for a in flat]

    in_specs = [pl.BlockSpec((1, 1, B), lambda s: (s, 0, 0)),
                pl.BlockSpec((1, K, H), lambda s: (s, 0, 0))]
    in_specs += [_full(a.shape) for a in flat]

    out_t, oh2d, loss = pl.pallas_call(
        _vq_step,
        grid=(S,),
        in_specs=in_specs,
        out_specs=[pl.BlockSpec((1, 1, B), lambda s: (s, 0, 0)),
                   pl.BlockSpec((B, K), lambda s: (0, s)),
                   pl.BlockSpec((1, 1), lambda s: (0, 0))],
        out_shape=[jax.ShapeDtypeStruct((S, 1, B), jnp.float32),
                   jax.ShapeDtypeStruct((B, S * K), jnp.float32),
                   jax.ShapeDtypeStruct((1, 1), jnp.float32)],
        compiler_params=pltpu.CompilerParams(
            dimension_semantics=("arbitrary",)),
    )(x_t, p['codebooks'], *flat)

    out = out_t.reshape(S, B).T
    onehot = oh2d.reshape(B, S, K)
    return out, loss[0, 0], onehot


# speedup vs baseline: 1.4104x; 1.1707x over previous
import jax
import jax.numpy as jnp
from jax.experimental import pallas as pl
from jax.experimental.pallas import tpu as pltpu

B, S, H, K = 512, 64, 256, 1024
H2, H4 = H // 2, H // 4


def _mm(a, w):
    return jax.lax.dot_general(a, w, (((1,), (0,)), ((), ())),
                               preferred_element_type=jnp.float32)


def _ln(h, g, b):
    m = jnp.mean(h, axis=-1, keepdims=True)
    v = jnp.mean((h - m) ** 2, axis=-1, keepdims=True)
    return (h - m) / jnp.sqrt(v + 1e-5) * g + b


def _vq_step(x_ref, cb_ref,
             eW1, eb1, eg1, ebn1, eW2, eb2, eg2, ebn2, eW3, eb3,
             er0W1, er0b1, er0W2, er0b2, er1W1, er1b1, er1W2, er1b2,
             dr0W1, dr0b1, dr0W2, dr0b2, dr1W1, dr1b1, dr1W2, dr1b2,
             dW1, db1, dg1, dbn1, dW2, db2, dg2, dbn2, dW3, db3,
             out_ref, oh_ref, loss_ref):
    s = pl.program_id(0) * 8 + pl.program_id(1)
    s_lo = pl.program_id(1)
    xrow = x_ref[0]
    xcol = xrow.T

    h = xcol * eW1[...] + eb1[...]
    h = jnp.tanh(_ln(h, eg1[...], ebn1[...]))
    h = jnp.tanh(_ln(_mm(h, eW2[...]) + eb2[...], eg2[...], ebn2[...]))
    h = _mm(h, eW3[...]) + eb3[...]
    for W1, b1, W2, b2 in ((er0W1, er0b1, er0W2, er0b2),
                           (er1W1, er1b1, er1W2, er1b2)):
        t = _mm(jax.nn.relu(h), W1[...]) + b1[...]
        t = _mm(jax.nn.relu(t), W2[...]) + b2[...]
        h = h + t
    h = jax.nn.relu(h)

    cb = cb_ref[0]
    x2 = jnp.sum(h * h, axis=-1, keepdims=True)
    e2 = jnp.sum(cb * cb, axis=-1)[None, :]
    xe = jax.lax.dot_general(h, cb, (((1,), (1,)), ((), ())),
                             preferred_element_type=jnp.float32)
    dist = x2 + e2 - 2.0 * xe
    mind = jnp.min(dist, axis=-1, keepdims=True)
    iota = jax.lax.broadcasted_iota(jnp.int32, (B, K), 1)
    idx = jnp.min(jnp.where(dist == mind, iota, K), axis=-1, keepdims=True)
    onehot = (iota == idx).astype(jnp.float32)
    oh_ref[:, pl.ds(s_lo, 1), :] = onehot[:, None, :]

    q = _mm(onehot, cb)
    commit_part = jnp.sum((q - h) ** 2)
    d = h + (q - h)

    for W1, b1, W2, b2 in ((dr0W1, dr0b1, dr0W2, dr0b2),
                           (dr1W1, dr1b1, dr1W2, dr1b2)):
        t = _mm(jax.nn.relu(d), W1[...]) + b1[...]
        t = _mm(jax.nn.relu(t), W2[...]) + b2[...]
        d = d + t
    d = jax.nn.relu(d)
    d = jnp.tanh(_ln(_mm(d, dW1[...]) + db1[...], dg1[...], dbn1[...]))
    d = jnp.tanh(_ln(_mm(d, dW2[...]) + db2[...], dg2[...], dbn2[...]))
    orow = jax.lax.dot_general(dW3[...], d, (((0,), (1,)), ((), ())),
                               preferred_element_type=jnp.float32) + db3[...]
    out_ref[0] = orow

    recons_part = jnp.sum((orow - xrow) ** 2)
    part = commit_part / (B * S * H) + recons_part / (B * S)

    @pl.when(s == 0)
    def _init():
        loss_ref[...] = jnp.zeros((1, 1), jnp.float32)
    loss_ref[...] += jnp.reshape(part, (1, 1))


def _full(shape):
    nd = len(shape)
    return pl.BlockSpec(shape, lambda i, j, _nd=nd: (0,) * _nd)


def kernel(x, params):
    p = params
    x_t = x.T.reshape(S, 1, B)

    def w2(a):
        return a.reshape(1, -1) if a.ndim == 1 else a

    flat = [p['eW1'], p['eb1'], p['eg1'], p['ebn1'],
            p['eW2'], p['eb2'], p['eg2'], p['ebn2'],
            p['eW3'], p['eb3']]
    for lay in p['enc_res']:
        flat.extend(lay)
    for lay in p['dec_res']:
        flat.extend(lay)
    flat.extend([p['dW1'], p['db1'], p['dg1'], p['dbn1'],
                 p['dW2'], p['db2'], p['dg2'], p['dbn2'],
                 p['dW3'], p['db3']])
    flat = [w2(a) for a in flat]

    in_specs = [pl.BlockSpec((1, 1, B), lambda i, j: (i * 8 + j, 0, 0)),
                pl.BlockSpec((1, K, H), lambda i, j: (i * 8 + j, 0, 0))]
    in_specs += [_full(a.shape) for a in flat]

    out_t, onehot, loss = pl.pallas_call(
        _vq_step,
        grid=(S // 8, 8),
        in_specs=in_specs,
        out_specs=[pl.BlockSpec((1, 1, B), lambda i, j: (i * 8 + j, 0, 0)),
                   pl.BlockSpec((B, 8, K), lambda i, j: (0, i, 0)),
                   pl.BlockSpec((1, 1), lambda i, j: (0, 0))],
        out_shape=[jax.ShapeDtypeStruct((S, 1, B), jnp.float32),
                   jax.ShapeDtypeStruct((B, S, K), jnp.float32),
                   jax.ShapeDtypeStruct((1, 1), jnp.float32)],
        compiler_params=pltpu.CompilerParams(
            dimension_semantics=("arbitrary", "arbitrary")),
    )(x_t, p['codebooks'], *flat)

    out = out_t.reshape(S, B).T
    return out, loss[0, 0], onehot


# speedup vs baseline: 1.6657x; 1.1811x over previous
import jax
import jax.numpy as jnp
from jax.experimental import pallas as pl
from jax.experimental.pallas import tpu as pltpu

B, S, H, K = 512, 64, 256, 1024
H2, H4 = H // 2, H // 4
ST = 8


def _mm(a, w):
    return jax.lax.dot_general(a, w, (((1,), (0,)), ((), ())),
                               preferred_element_type=jnp.float32)


def _ln(h, g, b):
    m = jnp.mean(h, axis=-1, keepdims=True)
    v = jnp.mean((h - m) ** 2, axis=-1, keepdims=True)
    return (h - m) / jnp.sqrt(v + 1e-5) * g + b


def _vq_step(x_ref, cb_ref,
             eW1, eb1, eg1, ebn1, eW2, eb2, eg2, ebn2, eW3, eb3,
             er0W1, er0b1, er0W2, er0b2, er1W1, er1b1, er1W2, er1b2,
             dr0W1, dr0b1, dr0W2, dr0b2, dr1W1, dr1b1, dr1W2, dr1b2,
             dW1, db1, dg1, dbn1, dW2, db2, dg2, dbn2, dW3, db3,
             out_ref, oh_ref, loss_ref):
    i = pl.program_id(0)
    part = jnp.zeros((1, 1), jnp.float32)

    for j in range(ST):
        xrow = x_ref[j]
        xcol = xrow.T

        h = xcol * eW1[...] + eb1[...]
        h = jnp.tanh(_ln(h, eg1[...], ebn1[...]))
        h = jnp.tanh(_ln(_mm(h, eW2[...]) + eb2[...], eg2[...], ebn2[...]))
        h = _mm(h, eW3[...]) + eb3[...]
        for W1, b1, W2, b2 in ((er0W1, er0b1, er0W2, er0b2),
                               (er1W1, er1b1, er1W2, er1b2)):
            t = _mm(jax.nn.relu(h), W1[...]) + b1[...]
            t = _mm(jax.nn.relu(t), W2[...]) + b2[...]
            h = h + t
        h = jax.nn.relu(h)

        cb = cb_ref[j]
        x2 = jnp.sum(h * h, axis=-1, keepdims=True)
        e2 = jnp.sum(cb * cb, axis=-1)[None, :]
        xe = jax.lax.dot_general(h, cb, (((1,), (1,)), ((), ())),
                                 preferred_element_type=jnp.float32)
        dist = x2 + e2 - 2.0 * xe
        mind = jnp.min(dist, axis=-1, keepdims=True)
        iota = jax.lax.broadcasted_iota(jnp.int32, (B, K), 1)
        idx = jnp.min(jnp.where(dist == mind, iota, K), axis=-1,
                      keepdims=True)
        onehot = (iota == idx).astype(jnp.float32)
        oh_ref[:, j, :] = onehot

        q = _mm(onehot, cb)
        commit_part = jnp.sum((q - h) ** 2)
        d = h + (q - h)

        for W1, b1, W2, b2 in ((dr0W1, dr0b1, dr0W2, dr0b2),
                               (dr1W1, dr1b1, dr1W2, dr1b2)):
            t = _mm(jax.nn.relu(d), W1[...]) + b1[...]
            t = _mm(jax.nn.relu(t), W2[...]) + b2[...]
            d = d + t
        d = jax.nn.relu(d)
        d = jnp.tanh(_ln(_mm(d, dW1[...]) + db1[...], dg1[...], dbn1[...]))
        d = jnp.tanh(_ln(_mm(d, dW2[...]) + db2[...], dg2[...], dbn2[...]))
        orow = jax.lax.dot_general(dW3[...], d, (((0,), (1,)), ((), ())),
                                   preferred_element_type=jnp.float32) \
            + db3[...]
        out_ref[j] = orow

        recons_part = jnp.sum((orow - xrow) ** 2)
        part = part + (commit_part / (B * S * H) + recons_part / (B * S))

    @pl.when(i == 0)
    def _init():
        loss_ref[...] = jnp.zeros((1, 1), jnp.float32)
    loss_ref[...] += part


def _full(shape):
    nd = len(shape)
    return pl.BlockSpec(shape, lambda i, _nd=nd: (0,) * _nd)


def kernel(x, params):
    p = params
    x_t = x.T.reshape(S, 1, B)

    def w2(a):
        return a.reshape(1, -1) if a.ndim == 1 else a

    flat = [p['eW1'], p['eb1'], p['eg1'], p['ebn1'],
            p['eW2'], p['eb2'], p['eg2'], p['ebn2'],
            p['eW3'], p['eb3']]
    for lay in p['enc_res']:
        flat.extend(lay)
    for lay in p['dec_res']:
        flat.extend(lay)
    flat.extend([p['dW1'], p['db1'], p['dg1'], p['dbn1'],
                 p['dW2'], p['db2'], p['dg2'], p['dbn2'],
                 p['dW3'], p['db3']])
    flat = [w2(a) for a in flat]

    in_specs = [pl.BlockSpec((ST, 1, B), lambda i: (i, 0, 0)),
                pl.BlockSpec((ST, K, H), lambda i: (i, 0, 0))]
    in_specs += [_full(a.shape) for a in flat]

    out_t, onehot, loss = pl.pallas_call(
        _vq_step,
        grid=(S // ST,),
        in_specs=in_specs,
        out_specs=[pl.BlockSpec((ST, 1, B), lambda i: (i, 0, 0)),
                   pl.BlockSpec((B, ST, K), lambda i: (0, i, 0)),
                   pl.BlockSpec((1, 1), lambda i: (0, 0))],
        out_shape=[jax.ShapeDtypeStruct((S, 1, B), jnp.float32),
                   jax.ShapeDtypeStruct((B, S, K), jnp.float32),
                   jax.ShapeDtypeStruct((1, 1), jnp.float32)],
        compiler_params=pltpu.CompilerParams(
            dimension_semantics=("arbitrary",)),
    )(x_t, p['codebooks'], *flat)

    out = out_t.reshape(S, B).T
    return out, loss[0, 0], onehot


# speedup vs baseline: 1.6693x; 1.0021x over previous
import jax
import jax.numpy as jnp
from jax.experimental import pallas as pl
from jax.experimental.pallas import tpu as pltpu

B, S, H, K = 512, 64, 256, 1024
H2, H4 = H // 2, H // 4
ST = 8


def _mm(a, w):
    return jax.lax.dot_general(a, w, (((1,), (0,)), ((), ())),
                               preferred_element_type=jnp.float32)


def _ln(h, g, b):
    m = jnp.mean(h, axis=-1, keepdims=True)
    v = jnp.mean((h - m) ** 2, axis=-1, keepdims=True)
    return (h - m) / jnp.sqrt(v + 1e-5) * g + b


def _vq_step(x_ref, cb_ref,
             eW1, eb1, eg1, ebn1, eW2, eb2, eg2, ebn2, eW3, eb3,
             er0W1, er0b1, er0W2, er0b2, er1W1, er1b1, er1W2, er1b2,
             dr0W1, dr0b1, dr0W2, dr0b2, dr1W1, dr1b1, dr1W2, dr1b2,
             dW1, db1, dg1, dbn1, dW2, db2, dg2, dbn2, dW3, db3,
             out_ref, oh_ref, loss_ref):
    i = pl.program_id(0)
    part = jnp.zeros((1, 1), jnp.float32)

    for j in range(ST):
        xrow = x_ref[j]
        xcol = xrow.T

        h = xcol * eW1[...] + eb1[...]
        h = jnp.tanh(_ln(h, eg1[...], ebn1[...]))
        h = jnp.tanh(_ln(_mm(h, eW2[...]) + eb2[...], eg2[...], ebn2[...]))
        h = _mm(h, eW3[...]) + eb3[...]
        for W1, b1, W2, b2 in ((er0W1, er0b1, er0W2, er0b2),
                               (er1W1, er1b1, er1W2, er1b2)):
            t = _mm(jax.nn.relu(h), W1[...]) + b1[...]
            t = _mm(jax.nn.relu(t), W2[...]) + b2[...]
            h = h + t
        h = jax.nn.relu(h)

        cb = cb_ref[j]
        x2 = jnp.sum(h * h, axis=-1, keepdims=True)
        e2 = jnp.sum(cb * cb, axis=-1)[None, :]
        xe = jax.lax.dot_general(h, cb, (((1,), (1,)), ((), ())),
                                 preferred_element_type=jnp.float32)
        dist = x2 + e2 - 2.0 * xe
        mind = jnp.min(dist, axis=-1, keepdims=True)
        iota = jax.lax.broadcasted_iota(jnp.int32, (B, K), 1)
        idx = jnp.min(jnp.where(dist == mind, iota, K), axis=-1,
                      keepdims=True)
        onehot = (iota == idx).astype(jnp.float32)
        oh_ref[:, j, :] = onehot

        q = _mm(onehot, cb)
        commit_part = jnp.sum((q - h) ** 2)
        d = h + (q - h)

        for W1, b1, W2, b2 in ((dr0W1, dr0b1, dr0W2, dr0b2),
                               (dr1W1, dr1b1, dr1W2, dr1b2)):
            t = _mm(jax.nn.relu(d), W1[...]) + b1[...]
            t = _mm(jax.nn.relu(t), W2[...]) + b2[...]
            d = d + t
        d = jax.nn.relu(d)
        d = jnp.tanh(_ln(_mm(d, dW1[...]) + db1[...], dg1[...], dbn1[...]))
        d = jnp.tanh(_ln(_mm(d, dW2[...]) + db2[...], dg2[...], dbn2[...]))
        orow = jax.lax.dot_general(dW3[...], d, (((0,), (1,)), ((), ())),
                                   preferred_element_type=jnp.float32) \
            + db3[...]
        out_ref[j] = orow

        recons_part = jnp.sum((orow - xrow) ** 2)
        part = part + (commit_part / (B * S * H) + recons_part / (B * S))

    del i
    loss_ref[...] = part[:, :, None]


def _full(shape):
    nd = len(shape)
    return pl.BlockSpec(shape, lambda i, _nd=nd: (0,) * _nd)


def kernel(x, params):
    p = params
    x_t = x.T.reshape(S, 1, B)

    def w2(a):
        return a.reshape(1, -1) if a.ndim == 1 else a

    flat = [p['eW1'], p['eb1'], p['eg1'], p['ebn1'],
            p['eW2'], p['eb2'], p['eg2'], p['ebn2'],
            p['eW3'], p['eb3']]
    for lay in p['enc_res']:
        flat.extend(lay)
    for lay in p['dec_res']:
        flat.extend(lay)
    flat.extend([p['dW1'], p['db1'], p['dg1'], p['dbn1'],
                 p['dW2'], p['db2'], p['dg2'], p['dbn2'],
                 p['dW3'], p['db3']])
    flat = [w2(a) for a in flat]

    in_specs = [pl.BlockSpec((ST, 1, B), lambda i: (i, 0, 0)),
                pl.BlockSpec((ST, K, H), lambda i: (i, 0, 0))]
    in_specs += [_full(a.shape) for a in flat]

    out_t, onehot, loss = pl.pallas_call(
        _vq_step,
        grid=(S // ST,),
        in_specs=in_specs,
        out_specs=[pl.BlockSpec((ST, 1, B), lambda i: (i, 0, 0)),
                   pl.BlockSpec((B, ST, K), lambda i: (0, i, 0)),
                   pl.BlockSpec((1, 1, 1), lambda i: (i, 0, 0))],
        out_shape=[jax.ShapeDtypeStruct((S, 1, B), jnp.float32),
                   jax.ShapeDtypeStruct((B, S, K), jnp.float32),
                   jax.ShapeDtypeStruct((S // ST, 1, 1), jnp.float32)],
        compiler_params=pltpu.CompilerParams(
            dimension_semantics=("parallel",)),
    )(x_t, p['codebooks'], *flat)

    out = out_t.reshape(S, B).T
    return out, jnp.sum(loss), onehot
